# baseline (speedup 1.0000x reference)
import functools

import jax
import jax.numpy as jnp
from jax import lax
from jax.experimental import pallas as pl
from jax.experimental.pallas import tpu as pltpu
from jax.experimental.pallas import tpu_sc as plsc

D_MODEL = 64
MAX_LEN = 2000
B = 4096
L = 200

NC = 2
NS = 16
NW = NC * NS

TOTAL = B * L
PER_W = TOTAL // NW
CHUNK = 640
NCHUNK = PER_W // CHUNK
NPAIR = NCHUNK // 2


@functools.partial(
    pl.kernel,
    out_type=jax.ShapeDtypeStruct((TOTAL, D_MODEL), jnp.float32),
    mesh=plsc.VectorSubcoreMesh(core_axis_name="c", subcore_axis_name="s"),
    scratch_types=[
        pltpu.VMEM((PER_W,), jnp.int32),
        pltpu.VMEM((CHUNK, D_MODEL), jnp.float32),
        pltpu.VMEM((CHUNK, D_MODEL), jnp.float32),
        pltpu.SemaphoreType.DMA,
        pltpu.SemaphoreType.DMA,
    ],
    compiler_params=pltpu.CompilerParams(use_tc_tiling_on_sc=False),
)
def _gather_kernel(pos_hbm, pe_hbm, out_hbm, idx_v, r0, r1, sg0, sg1):
    wid = lax.axis_index("s") * NC + lax.axis_index("c")
    base = wid * PER_W
    pltpu.sync_copy(pos_hbm.at[pl.ds(base, PER_W)], idx_v)

    def g_start(c, rbuf, sem):
        pltpu.async_copy(pe_hbm.at[idx_v.at[pl.ds(c * CHUNK, CHUNK)]], rbuf, sem)

    def g_wait(rbuf, sem):
        pltpu.make_async_copy(pe_hbm.at[idx_v.at[pl.ds(0, CHUNK)]], rbuf, sem).wait()

    g_start(0, r0, sg0)

    def pair_body(p, carry):
        c0 = 2 * p
        g_start(c0 + 1, r1, sg1)
        g_wait(r0, sg0)
        pltpu.sync_copy(r0, out_hbm.at[pl.ds(base + c0 * CHUNK, CHUNK)])

        @pl.when(p < NPAIR - 1)
        def _():
            g_start(c0 + 2, r0, sg0)

        g_wait(r1, sg1)
        pltpu.sync_copy(r1, out_hbm.at[pl.ds(base + (c0 + 1) * CHUNK, CHUNK)])
        return carry

    lax.fori_loop(0, NPAIR, pair_body, 0)


def kernel(positions, pe):
    flat = positions.reshape(TOTAL).astype(jnp.int32)
    out = _gather_kernel(flat, pe)
    return out.reshape(B, L, D_MODEL)


# speedup vs baseline: 1.0003x; 1.0003x over previous
import functools

import jax
import jax.numpy as jnp
from jax import lax
from jax.experimental import pallas as pl
from jax.experimental.pallas import tpu as pltpu
from jax.experimental.pallas import tpu_sc as plsc

D_MODEL = 64
MAX_LEN = 2000
B = 4096
L = 200

NC = 2
NS = 16
NW = NC * NS

TOTAL = B * L
PER_W = TOTAL // NW
BAT_W = B // NW
NB = 2
NG = BAT_W // NB
NPAIR = NG // 2


@functools.partial(
    pl.kernel,
    out_type=jax.ShapeDtypeStruct((B, L, D_MODEL), jnp.float32),
    mesh=plsc.VectorSubcoreMesh(core_axis_name="c", subcore_axis_name="s"),
    scratch_types=[
        pltpu.VMEM((PER_W,), jnp.int32),
        pltpu.VMEM((NB, L, D_MODEL), jnp.float32),
        pltpu.VMEM((NB, L, D_MODEL), jnp.float32),
        pltpu.SemaphoreType.DMA,
        pltpu.SemaphoreType.DMA,
    ],
    compiler_params=pltpu.CompilerParams(use_tc_tiling_on_sc=False),
)
def _gather_kernel(pos_hbm, pe_hbm, out_hbm, idx_v, r0, r1, sg0, sg1):
    wid = lax.axis_index("s") * NC + lax.axis_index("c")
    base = wid * PER_W
    wb = wid * BAT_W
    pltpu.sync_copy(pos_hbm.at[pl.ds(base, PER_W)], idx_v)

    def g_start(g, rbuf, sem):
        for j in range(NB):
            pltpu.async_copy(
                pe_hbm.at[idx_v.at[pl.ds(g * (NB * L) + j * L, L)]],
                rbuf.at[j], sem)

    def g_wait(rbuf, sem):
        for j in range(NB):
            pltpu.make_async_copy(
                pe_hbm.at[idx_v.at[pl.ds(0, L)]], rbuf.at[j], sem).wait()

    g_start(0, r0, sg0)

    def pair_body(p, carry):
        g0 = 2 * p
        g_start(g0 + 1, r1, sg1)
        g_wait(r0, sg0)
        pltpu.sync_copy(r0, out_hbm.at[pl.ds(wb + g0 * NB, NB)])

        @pl.when(p < NPAIR - 1)
        def _():
            g_start(g0 + 2, r0, sg0)

        g_wait(r1, sg1)
        pltpu.sync_copy(r1, out_hbm.at[pl.ds(wb + (g0 + 1) * NB, NB)])
        return carry

    lax.fori_loop(0, NPAIR, pair_body, 0)


def kernel(positions, pe):
    flat = positions.reshape(TOTAL).astype(jnp.int32)
    return _gather_kernel(flat, pe)
